# baseline (speedup 1.0000x reference)
import functools

import jax
import jax.numpy as jnp
from jax import lax
from jax.experimental import pallas as pl
from jax.experimental.pallas import tpu as pltpu
from jax.experimental.pallas import tpu_sc as plsc

_K = 64
_N = 32768
_NV = _N // 16
_CAP = 2048
_ROWS = 128
_NW = 32
_RPW = _ROWS // _NW


def _enc_s(f):
    bi = lax.bitcast_convert_type(f, jnp.int32)
    return jnp.where(bi >= 0, bi, jnp.int32(-2147483648) - bi)


def _dec_s(e):
    bi = jnp.where(e >= 0, e, jnp.int32(-2147483648) - e)
    return lax.bitcast_convert_type(bi, jnp.float32)


def _log2_approx(c):
    fl = lax.bitcast_convert_type(c.astype(jnp.float32), jnp.int32)
    return (fl - 0x3F800000).astype(jnp.float32) * jnp.float32(1.0 / (1 << 23))


def _scal_max(v):
    s = v[0]
    for i in range(1, 16):
        s = jnp.maximum(s, v[i])
    return s


def _scal_min(v):
    s = v[0]
    for i in range(1, 16):
        s = jnp.minimum(s, v[i])
    return s


def _recip(d):
    s = jnp.where(d < 0, jnp.float32(-1.0), jnp.float32(1.0))
    a = jnp.abs(d)
    r = lax.bitcast_convert_type(
        jnp.int32(0x7EF311C3) - lax.bitcast_convert_type(a, jnp.int32),
        jnp.float32)
    r = r * (2.0 - a * r)
    r = r * (2.0 - a * r)
    return s * r


def _sc_body(x_hbm, o_hbm, row_v, cand_v):
    wid = lax.axis_index("s") * 2 + lax.axis_index("c")

    def do_row(rr, _):
        r = wid * _RPW + rr
        pltpu.sync_copy(x_hbm.at[r], row_v)

        ninf = jnp.full((16,), -jnp.inf, jnp.float32)

        def body_a(j, ms):
            m0, m1, m2, m3 = ms
            b = j * 64
            m0 = jnp.maximum(m0, row_v[pl.ds(b, 16)])
            m1 = jnp.maximum(m1, row_v[pl.ds(b + 16, 16)])
            m2 = jnp.maximum(m2, row_v[pl.ds(b + 32, 16)])
            m3 = jnp.maximum(m3, row_v[pl.ds(b + 48, 16)])
            return m0, m1, m2, m3

        m0, m1, m2, m3 = lax.fori_loop(0, _NV // 4, body_a,
                                       (ninf, ninf, ninf, ninf), unroll=4)
        mm = jnp.minimum(jnp.minimum(m0, m1), jnp.minimum(m2, m3))
        mx = jnp.maximum(jnp.maximum(m0, m1), jnp.maximum(m2, m3))
        lo0 = _scal_min(mm)
        hi0 = _dec_s(_enc_s(_scal_max(mx)) + 1)

        def body_clr(j, _):
            cand_v[pl.ds(j * 16, 16)] = ninf
            return 0

        lax.fori_loop(0, _CAP // 16, body_clr, 0, unroll=8)

        lo0_splat = jnp.full((16,), lo0, jnp.float32)

        def body_b(j, ptr):
            v = row_v[pl.ds(j * 16, 16)]
            msk = v >= lo0_splat
            pc = plsc.all_reduce_population_count(msk)
            plsc.store_compressed(
                cand_v.at[pl.ds(jnp.minimum(ptr, _CAP - 16), 16)], v,
                mask=msk)
            return ptr + pc[0]

        n0 = lax.fori_loop(0, _NV, body_b, jnp.int32(0), unroll=8)

        def count_pass(cand):
            cs = jnp.full((16,), cand, jnp.float32)
            inf = jnp.full((16,), jnp.inf, jnp.float32)

            def body_c(j, st):
                acc, mn, mxx = st
                v = cand_v[pl.ds(j * 16, 16)]
                km = v >= cs
                acc = acc + plsc.all_reduce_population_count(km)
                mn = jnp.minimum(mn, jnp.where(km, v, inf))
                mxx = jnp.maximum(mxx, jnp.where(km, -inf, v))
                return acc, mn, mxx

            acc, mn, mxx = lax.fori_loop(
                0, _CAP // 16, body_c,
                (jnp.zeros((16,), jnp.int32), inf, -inf), unroll=8)
            return acc[0], _scal_min(mn), _scal_max(mxx)

        def cond(st):
            it, lo, hi, clo, v1, l1, v0, l0 = st
            return ((_enc_s(hi) - _enc_s(lo) > 1) & (clo != _K) & (it < 16))

        def body_w(st):
            it, lo, hi, clo, v1, l1, v0, l0 = st
            denom = l0 - l1
            degen = (jnp.abs(denom) < 1e-6) | (v0 == v1)
            cand = v1 + (6.0 - l1) * (v0 - v1) * _recip(
                jnp.where(degen, 1.0, denom))
            el, eh = _enc_s(lo), _enc_s(hi)
            ce = jnp.where(degen, el + ((eh - el) >> 1), _enc_s(cand))
            ce = jnp.minimum(jnp.maximum(ce, el + 1), eh - 1)
            cand = _dec_s(ce)
            c, smin, mlt = count_pass(cand)
            lc = _log2_approx(jnp.maximum(c, 1))
            ge = c >= _K
            lo = jnp.where(ge, smin, lo)
            clo = jnp.where(ge, c, clo)
            hi = jnp.where(ge, hi, _dec_s(_enc_s(mlt) + 1))
            newv = jnp.where(ge, smin, mlt)
            newl = jnp.where(ge, lc, _log2_approx(c + 1))
            return it + 1, lo, hi, clo, newv, newl, v1, l1

        st0 = (jnp.int32(0), lo0, hi0, n0, lo0,
               _log2_approx(jnp.maximum(n0, 1)), hi0, jnp.float32(-1.0))
        _, t, _, _, _, _, _, _ = lax.while_loop(cond, body_w, st0)

        ts = jnp.full((16,), t, jnp.float32)
        zero = jnp.zeros((16,), jnp.float32)

        def body_d(j, _):
            v = row_v[pl.ds(j * 16, 16)]
            row_v[pl.ds(j * 16, 16)] = jnp.where(v >= ts, v, zero)
            return 0

        lax.fori_loop(0, _NV, body_d, 0, unroll=8)
        pltpu.sync_copy(row_v, o_hbm.at[r])
        return 0

    lax.fori_loop(0, _RPW, do_row, 0)


def sc_topk_mask(x):
    mesh = plsc.VectorSubcoreMesh(core_axis_name="c", subcore_axis_name="s")
    k = functools.partial(
        pl.kernel,
        mesh=mesh,
        out_type=jax.ShapeDtypeStruct((_ROWS, _N), jnp.float32),
        scratch_types=[
            pltpu.VMEM((_N,), jnp.float32),
            pltpu.VMEM((_CAP,), jnp.float32),
        ],
        compiler_params=pltpu.CompilerParams(needs_layout_passes=False),
    )(_sc_body)
    return k(x)


def kernel(x):
    return sc_topk_mask(x)


# speedup vs baseline: 2.6983x; 2.6983x over previous
import jax
import jax.numpy as jnp
from jax.experimental import pallas as pl
from jax.experimental.pallas import tpu as pltpu

_K = 64
_R = 32
_N = 32768
_W = 512
_NS = _N // _W


def _enc(f):
    bi = jax.lax.bitcast_convert_type(f, jnp.int32)
    return jnp.where(bi >= 0, bi, jnp.int32(-2147483648) - bi)


def _dec(e):
    bi = jnp.where(e >= 0, e, jnp.int32(-2147483648) - e)
    return jax.lax.bitcast_convert_type(bi, jnp.float32)


def _pass(ref, cand, ns):
    inf = jnp.float32(jnp.inf)
    xs = ref[:, 0:_W]
    km = xs >= cand
    acc_c = km.astype(jnp.int32)
    acc_mn = jnp.where(km, xs, inf)
    acc_mx = jnp.where(km, -inf, xs)
    for k in range(1, ns):
        xs = ref[:, k * _W:(k + 1) * _W]
        km = xs >= cand
        acc_c = acc_c + km.astype(jnp.int32)
        acc_mn = jnp.minimum(acc_mn, jnp.where(km, xs, inf))
        acc_mx = jnp.maximum(acc_mx, jnp.where(km, -inf, xs))
    c = jnp.sum(acc_c, axis=1, keepdims=True)
    smin = jnp.min(acc_mn, axis=1, keepdims=True)
    mlt = jnp.max(acc_mx, axis=1, keepdims=True)
    return c, smin, mlt


def _refine(ref, ns, lo, hi, clo, v1, l1, v0, l0, maxit):
    def _open(lo, hi, clo):
        return (_enc(hi) - _enc(lo) > 1) & (clo != _K)

    def cond(st):
        i, lo, hi, clo, v1, l1, v0, l0 = st
        return (i < maxit) & jnp.any(_open(lo, hi, clo))

    def body(st):
        i, lo, hi, clo, v1, l1, v0, l0 = st
        is_open = _open(lo, hi, clo)
        el, eh = _enc(lo), _enc(hi)
        denom = l0 - l1
        degen = (jnp.abs(denom) < 1e-6) | (v0 == v1)
        cand_sec = v1 + (6.0 - l1) * (v0 - v1) / jnp.where(degen, 1.0, denom)
        ce = jnp.where(degen, el + (eh - el) // 2, _enc(cand_sec))
        ce = jnp.minimum(jnp.maximum(ce, el + 1), eh - 1)
        cand = _dec(ce)

        c, smin, mlt = _pass(ref, cand, ns)
        lc = jnp.log2(jnp.maximum(c.astype(jnp.float32), 0.5))
        ge = is_open & (c >= _K)
        lt = is_open & (c < _K)
        lo = jnp.where(ge, smin, lo)
        clo = jnp.where(ge, c, clo)
        hi = jnp.where(lt, _dec(_enc(mlt) + 1), hi)
        newv = jnp.where(ge, smin, mlt)
        newl = jnp.where(ge, lc, jnp.log2((c + 1).astype(jnp.float32)))
        v0 = jnp.where(is_open, v1, v0)
        l0 = jnp.where(is_open, l1, l0)
        v1 = jnp.where(is_open, newv, v1)
        l1 = jnp.where(is_open, newl, l1)
        return i + 1, lo, hi, clo, v1, l1, v0, l0

    st = jax.lax.while_loop(cond, body,
                            (jnp.int32(0), lo, hi, clo, v1, l1, v0, l0))
    return st[1]


def _topk_mask_block(x_ref, o_ref, m_ref):
    m = x_ref[:, 0:_W]
    for k in range(1, _NS):
        m = jnp.maximum(m, x_ref[:, k * _W:(k + 1) * _W])
    m_ref[...] = m
    g = m[:, 0:64]
    for k in range(1, 8):
        g = jnp.maximum(g, m[:, k * 64:(k + 1) * 64])
    lo0 = jnp.min(g, axis=1, keepdims=True)
    hi0 = _dec(_enc(jnp.max(g, axis=1, keepdims=True)) + 1)

    cm0, smin_m, _ = _pass(m_ref, lo0, 1)
    lm1 = jnp.log2(cm0.astype(jnp.float32))
    lm0 = jnp.full((_R, 1), -1.0, dtype=jnp.float32)
    m64 = _refine(m_ref, 1, smin_m, hi0, cm0, smin_m, lm1, hi0, lm0, 12)

    c0, smin0, _ = _pass(x_ref, m64, _NS)
    l1 = jnp.log2(c0.astype(jnp.float32))
    l0 = jnp.full((_R, 1), -1.0, dtype=jnp.float32)
    t = _refine(x_ref, _NS, smin0, hi0, c0, smin0, l1, hi0, l0, 16)

    x = x_ref[...]
    o_ref[...] = jnp.where(x >= t, x, jnp.float32(0.0))


def kernel(x):
    rows, cols = x.shape
    grid = rows // _R
    return pl.pallas_call(
        _topk_mask_block,
        grid=(grid,),
        in_specs=[pl.BlockSpec((_R, cols), lambda i: (i, 0))],
        out_specs=pl.BlockSpec((_R, cols), lambda i: (i, 0)),
        out_shape=jax.ShapeDtypeStruct(x.shape, x.dtype),
        scratch_shapes=[pltpu.VMEM((_R, _W), jnp.float32)],
    )(x)
